# baseline (speedup 1.0000x reference)
import numpy as np
import jax
import jax.numpy as jnp
from jax.experimental import pallas as pl

_B, _C, _D, _H, _W = 2, 384, 32, 16, 16
_HW = _H * _W
_S = _D * _HW
_HP = 6
_DC = 8
_ND = _D // _DC
_CC = 192
_NC = _C // _CC
_SC = 2048
_NS = _S // _SC


def _pe_dc() -> np.ndarray:
    pos = np.arange(_D, dtype=np.float32)[:, None]
    div = np.exp(np.arange(0, _C, 2, dtype=np.float32) * (-np.log(10000.0) / _C))
    pe = np.zeros((_D, _C), np.float32)
    pe[:, 0::2] = np.sin(pos * div)
    pe[:, 1::2] = np.cos(pos * div)
    return pe


def _qv_kernel(f_ref, pe_ref, wq_ref, wv_ref, v_ref, qp_ref):
    x = (f_ref[0] + pe_ref[0].T[:, :, None]).astype(jnp.bfloat16)
    q = jax.lax.dot_general(wq_ref[...], x, (((1,), (0,)), ((), ())),
                            preferred_element_type=jnp.float32)
    v = jax.lax.dot_general(wv_ref[...], x, (((1,), (0,)), ((), ())),
                            preferred_element_type=jnp.float32)
    v_ref[0] = v.astype(jnp.bfloat16)
    qp_ref[0] = q.max(axis=2).T


def _mix_kernel(v_ref, qp_ref, woff_ref, boff_ref, watt_ref, batt_ref,
                wo_ref, out_ref, st_ref):
    ci = pl.program_id(1)
    qp = qp_ref[0]
    off = jax.lax.dot_general(qp, woff_ref[...], (((1,), (1,)), ((), ())),
                              preferred_element_type=jnp.float32) + boff_ref[...]
    att = jax.lax.dot_general(qp, watt_ref[...], (((1,), (1,)), ((), ())),
                              preferred_element_type=jnp.float32) + batt_ref[...]
    att = jax.nn.softmax(att, axis=-1)
    off = jnp.clip(off, 0.0, float(_D - 1))
    low = jnp.floor(off)
    frac = off - low
    lowi = low.astype(jnp.int32)
    upi = jnp.ceil(off).astype(jnp.int32)
    dio = jax.lax.broadcasted_iota(jnp.int32, (_D, _D), 1)
    m = jnp.zeros((_D, _D), jnp.float32)
    for p in range(_HP):
        a = att[:, p][:, None]
        fr = frac[:, p][:, None]
        l = lowi[:, p][:, None]
        u = upi[:, p][:, None]
        m = m + a * ((1.0 - fr) * (dio == l).astype(jnp.float32)
                     + fr * (dio == u).astype(jnp.float32))
    s = jax.lax.dot_general(m.astype(jnp.bfloat16), v_ref[0],
                            (((1,), (1,)), ((), ())),
                            preferred_element_type=jnp.float32)
    part = jax.lax.dot_general(wo_ref[0], s.astype(jnp.bfloat16),
                               (((1,), (1,)), ((), ())),
                               preferred_element_type=jnp.float32)

    @pl.when(ci == 0)
    def _():
        out_ref[0] = part.astype(jnp.bfloat16)

    @pl.when(ci != 0)
    def _():
        out_ref[0] = (out_ref[0].astype(jnp.float32) + part).astype(jnp.bfloat16)

    @pl.when(ci == _NC - 1)
    def _():
        t = out_ref[0].astype(jnp.float32)
        st_ref[0, 0] = jnp.sum(t, axis=(1, 2))
        st_ref[0, 1] = jnp.sum(t * t, axis=(1, 2))


def _bn_kernel(op_ref, f_ref, st_ref, g_ref, b_ref, y_ref):
    n = float(_B * _S)
    ssum = st_ref[0, 0] + st_ref[1, 0]
    ssq = st_ref[0, 1] + st_ref[1, 1]
    mean = ssum / n
    var = ssq / n - mean * mean
    a = g_ref[0] * jax.lax.rsqrt(var + 1e-5)
    bb = b_ref[0] - mean * a
    y_ref[0] = (a[:, None] * op_ref[0].astype(jnp.float32)
                + bb[:, None] + f_ref[0])


def kernel(features, Wq, Wv, Wo, W_off, b_off, W_att, b_att, gamma, beta):
    f4 = features.reshape(_B, _C, _D, _HW)
    pe_r = jnp.asarray(_pe_dc()).reshape(_ND, _DC, _C)

    v, qp = pl.pallas_call(
        _qv_kernel,
        grid=(_B, _ND),
        in_specs=[
            pl.BlockSpec((1, _C, _DC, _HW), lambda b, d: (b, 0, d, 0)),
            pl.BlockSpec((1, _DC, _C), lambda b, d: (d, 0, 0)),
            pl.BlockSpec((_C, _C), lambda b, d: (0, 0)),
            pl.BlockSpec((_C, _C), lambda b, d: (0, 0)),
        ],
        out_specs=[
            pl.BlockSpec((1, _C, _DC, _HW), lambda b, d: (b, 0, d, 0)),
            pl.BlockSpec((1, _DC, _C), lambda b, d: (b, d, 0)),
        ],
        out_shape=[
            jax.ShapeDtypeStruct((_B, _C, _D, _HW), jnp.bfloat16),
            jax.ShapeDtypeStruct((_B, _D, _C), jnp.float32),
        ],
    )(f4, pe_r, Wq.astype(jnp.bfloat16), Wv.astype(jnp.bfloat16))

    wo_r = Wo.reshape(_C, _NC, _CC).transpose(1, 0, 2).astype(jnp.bfloat16)
    boff = b_off.reshape(1, _HP)
    batt = b_att.reshape(1, _HP)

    out_pre, st = pl.pallas_call(
        _mix_kernel,
        grid=(_B, _NC),
        in_specs=[
            pl.BlockSpec((1, _CC, _D, _HW), lambda b, c: (b, c, 0, 0)),
            pl.BlockSpec((1, _D, _C), lambda b, c: (b, 0, 0)),
            pl.BlockSpec((_HP, _C), lambda b, c: (0, 0)),
            pl.BlockSpec((1, _HP), lambda b, c: (0, 0)),
            pl.BlockSpec((_HP, _C), lambda b, c: (0, 0)),
            pl.BlockSpec((1, _HP), lambda b, c: (0, 0)),
            pl.BlockSpec((1, _C, _CC), lambda b, c: (c, 0, 0)),
        ],
        out_specs=[
            pl.BlockSpec((1, _C, _D, _HW), lambda b, c: (b, 0, 0, 0)),
            pl.BlockSpec((1, 2, _C), lambda b, c: (b, 0, 0)),
        ],
        out_shape=[
            jax.ShapeDtypeStruct((_B, _C, _D, _HW), jnp.bfloat16),
            jax.ShapeDtypeStruct((_B, 2, _C), jnp.float32),
        ],
    )(v, qp, W_off, boff, W_att, batt, wo_r)

    f3 = features.reshape(_B, _C, _S)
    op3 = out_pre.reshape(_B, _C, _S)

    y = pl.pallas_call(
        _bn_kernel,
        grid=(_B, _NS),
        in_specs=[
            pl.BlockSpec((1, _C, _SC), lambda b, s: (b, 0, s)),
            pl.BlockSpec((1, _C, _SC), lambda b, s: (b, 0, s)),
            pl.BlockSpec((_B, 2, _C), lambda b, s: (0, 0, 0)),
            pl.BlockSpec((1, _C), lambda b, s: (0, 0)),
            pl.BlockSpec((1, _C), lambda b, s: (0, 0)),
        ],
        out_specs=pl.BlockSpec((1, _C, _SC), lambda b, s: (b, 0, s)),
        out_shape=jax.ShapeDtypeStruct((_B, _C, _S), jnp.float32),
    )(op3, f3, st, gamma.reshape(1, _C), beta.reshape(1, _C))

    return y.reshape(_B, _C, _D, _H, _W)


# speedup vs baseline: 1.0664x; 1.0664x over previous
import numpy as np
import jax
import jax.numpy as jnp
from jax.experimental import pallas as pl

_B, _C, _D, _H, _W = 2, 384, 32, 16, 16
_HW = _H * _W
_S = _D * _HW
_HP = 6
_DC = 8
_ND = _D // _DC
_SC = 2048
_NS = _S // _SC


def _pe_dc() -> np.ndarray:
    pos = np.arange(_D, dtype=np.float32)[:, None]
    div = np.exp(np.arange(0, _C, 2, dtype=np.float32) * (-np.log(10000.0) / _C))
    pe = np.zeros((_D, _C), np.float32)
    pe[:, 0::2] = np.sin(pos * div)
    pe[:, 1::2] = np.cos(pos * div)
    return pe


def _fused_kernel(f_ref, pe_ref, wq_ref, wv_ref, woff_ref, boff_ref,
                  watt_ref, batt_ref, wo_ref, out_ref, st_ref):
    vs = []
    qps = []
    for dc in range(_ND):
        sl = slice(dc * _DC, (dc + 1) * _DC)
        x = (f_ref[0, :, sl, :] + pe_ref[sl].T[:, :, None]).astype(jnp.bfloat16)
        q = jax.lax.dot_general(wq_ref[...], x, (((1,), (0,)), ((), ())),
                                preferred_element_type=jnp.float32)
        qps.append(q.max(axis=2))
        v = jax.lax.dot_general(wv_ref[...], x, (((1,), (0,)), ((), ())),
                                preferred_element_type=jnp.float32)
        vs.append(v.astype(jnp.bfloat16))
    v = jnp.concatenate(vs, axis=1)
    qp = jnp.concatenate(qps, axis=1).T

    off = jax.lax.dot_general(qp, woff_ref[...], (((1,), (1,)), ((), ())),
                              preferred_element_type=jnp.float32) + boff_ref[...]
    att = jax.lax.dot_general(qp, watt_ref[...], (((1,), (1,)), ((), ())),
                              preferred_element_type=jnp.float32) + batt_ref[...]
    att = jax.nn.softmax(att, axis=-1)
    off = jnp.clip(off, 0.0, float(_D - 1))
    low = jnp.floor(off)
    frac = off - low
    lowi = low.astype(jnp.int32)
    upi = jnp.ceil(off).astype(jnp.int32)
    dio = jax.lax.broadcasted_iota(jnp.int32, (_D, _D), 1)
    m = jnp.zeros((_D, _D), jnp.float32)
    for p in range(_HP):
        a = att[:, p][:, None]
        fr = frac[:, p][:, None]
        l = lowi[:, p][:, None]
        u = upi[:, p][:, None]
        m = m + a * ((1.0 - fr) * (dio == l).astype(jnp.float32)
                     + fr * (dio == u).astype(jnp.float32))
    s = jax.lax.dot_general(m.astype(jnp.bfloat16), v,
                            (((1,), (1,)), ((), ())),
                            preferred_element_type=jnp.float32)
    part = jax.lax.dot_general(wo_ref[...], s.astype(jnp.bfloat16),
                               (((1,), (1,)), ((), ())),
                               preferred_element_type=jnp.float32)
    out_ref[0] = part.astype(jnp.bfloat16)
    st_ref[0, 0] = jnp.sum(part, axis=(1, 2))
    st_ref[0, 1] = jnp.sum(part * part, axis=(1, 2))


def _bn_kernel(op_ref, f_ref, st_ref, g_ref, b_ref, y_ref):
    n = float(_B * _S)
    ssum = st_ref[0, 0] + st_ref[1, 0]
    ssq = st_ref[0, 1] + st_ref[1, 1]
    mean = ssum / n
    var = ssq / n - mean * mean
    a = g_ref[0] * jax.lax.rsqrt(var + 1e-5)
    bb = b_ref[0] - mean * a
    y_ref[0] = (a[:, None] * op_ref[0].astype(jnp.float32)
                + bb[:, None] + f_ref[0])


def kernel(features, Wq, Wv, Wo, W_off, b_off, W_att, b_att, gamma, beta):
    f4 = features.reshape(_B, _C, _D, _HW)
    pe = jnp.asarray(_pe_dc())
    boff = b_off.reshape(1, _HP)
    batt = b_att.reshape(1, _HP)

    out_pre, st = pl.pallas_call(
        _fused_kernel,
        grid=(_B,),
        in_specs=[
            pl.BlockSpec((1, _C, _D, _HW), lambda b: (b, 0, 0, 0)),
            pl.BlockSpec((_D, _C), lambda b: (0, 0)),
            pl.BlockSpec((_C, _C), lambda b: (0, 0)),
            pl.BlockSpec((_C, _C), lambda b: (0, 0)),
            pl.BlockSpec((_HP, _C), lambda b: (0, 0)),
            pl.BlockSpec((1, _HP), lambda b: (0, 0)),
            pl.BlockSpec((_HP, _C), lambda b: (0, 0)),
            pl.BlockSpec((1, _HP), lambda b: (0, 0)),
            pl.BlockSpec((_C, _C), lambda b: (0, 0)),
        ],
        out_specs=[
            pl.BlockSpec((1, _C, _D, _HW), lambda b: (b, 0, 0, 0)),
            pl.BlockSpec((1, 2, _C), lambda b: (b, 0, 0)),
        ],
        out_shape=[
            jax.ShapeDtypeStruct((_B, _C, _D, _HW), jnp.bfloat16),
            jax.ShapeDtypeStruct((_B, 2, _C), jnp.float32),
        ],
    )(f4, pe, Wq.astype(jnp.bfloat16), Wv.astype(jnp.bfloat16),
      W_off, boff, W_att, batt, Wo.astype(jnp.bfloat16))

    f3 = features.reshape(_B, _C, _S)
    op3 = out_pre.reshape(_B, _C, _S)

    y = pl.pallas_call(
        _bn_kernel,
        grid=(_B, _NS),
        in_specs=[
            pl.BlockSpec((1, _C, _SC), lambda b, s: (b, 0, s)),
            pl.BlockSpec((1, _C, _SC), lambda b, s: (b, 0, s)),
            pl.BlockSpec((_B, 2, _C), lambda b, s: (0, 0, 0)),
            pl.BlockSpec((1, _C), lambda b, s: (0, 0)),
            pl.BlockSpec((1, _C), lambda b, s: (0, 0)),
        ],
        out_specs=pl.BlockSpec((1, _C, _SC), lambda b, s: (b, 0, s)),
        out_shape=jax.ShapeDtypeStruct((_B, _C, _S), jnp.float32),
    )(op3, f3, st, gamma.reshape(1, _C), beta.reshape(1, _C))

    return y.reshape(_B, _C, _D, _H, _W)


# speedup vs baseline: 1.4873x; 1.3946x over previous
import numpy as np
import jax
import jax.numpy as jnp
from jax.experimental import pallas as pl
from jax.experimental.pallas import tpu as pltpu

_B, _C, _D, _H, _W = 2, 384, 32, 16, 16
_HW = _H * _W
_S = _D * _HW
_HP = 6
_DC = 8
_ND = _D // _DC
_ZC = 8
_NZ = _D // _ZC
_CSTEPS = _B * _ND
_STEPS = 2 * _CSTEPS


def _pe_dc() -> np.ndarray:
    pos = np.arange(_D, dtype=np.float32)[:, None]
    div = np.exp(np.arange(0, _C, 2, dtype=np.float32) * (-np.log(10000.0) / _C))
    pe = np.zeros((_D, _C), np.float32)
    pe[:, 0::2] = np.sin(pos * div)
    pe[:, 1::2] = np.cos(pos * div)
    return pe


def _fused_kernel(f_ref, pe_ref, wq_ref, wv_ref, woff_ref, boff_ref,
                  watt_ref, batt_ref, wo_ref, g_ref, b_ref,
                  y_ref, v_scr, qp_scr, op_scr, st_scr):
    s_id = pl.program_id(0)
    b = (s_id // _ND) % _B
    d = s_id % _ND

    @pl.when(s_id < _CSTEPS)
    def _project():
        x = (f_ref[0] + pe_ref[pl.ds(d * _DC, _DC)].T[:, :, None]).astype(jnp.bfloat16)
        q = jax.lax.dot_general(wq_ref[...], x, (((1,), (0,)), ((), ())),
                                preferred_element_type=jnp.float32)
        v = jax.lax.dot_general(wv_ref[...], x, (((1,), (0,)), ((), ())),
                                preferred_element_type=jnp.float32)
        v_scr[:, pl.ds(d * _DC, _DC), :] = v.astype(jnp.bfloat16)
        qp_scr[pl.ds(d * _DC, _DC), :] = q.max(axis=2).T

    @pl.when((s_id < _CSTEPS) & (d == _ND - 1))
    def _mix():
        qp = qp_scr[...]
        off = jax.lax.dot_general(qp, woff_ref[...], (((1,), (1,)), ((), ())),
                                  preferred_element_type=jnp.float32) + boff_ref[...]
        att = jax.lax.dot_general(qp, watt_ref[...], (((1,), (1,)), ((), ())),
                                  preferred_element_type=jnp.float32) + batt_ref[...]
        att = jax.nn.softmax(att, axis=-1)
        off = jnp.clip(off, 0.0, float(_D - 1))
        low = jnp.floor(off)
        frac = off - low
        lowi = low.astype(jnp.int32)
        upi = jnp.ceil(off).astype(jnp.int32)
        dio = jax.lax.broadcasted_iota(jnp.int32, (_D, _D), 1)
        m = jnp.zeros((_D, _D), jnp.float32)
        for p in range(_HP):
            a = att[:, p][:, None]
            fr = frac[:, p][:, None]
            l = lowi[:, p][:, None]
            u = upi[:, p][:, None]
            m = m + a * ((1.0 - fr) * (dio == l).astype(jnp.float32)
                         + fr * (dio == u).astype(jnp.float32))
        m16 = m.astype(jnp.bfloat16)
        v = v_scr[...]
        ssum = jnp.zeros((_C,), jnp.float32)
        ssq = jnp.zeros((_C,), jnp.float32)
        for zc in range(_NZ):
            zsl = slice(zc * _ZC, (zc + 1) * _ZC)
            sc = jax.lax.dot_general(m16[zsl], v, (((1,), (1,)), ((), ())),
                                     preferred_element_type=jnp.float32)
            part = jax.lax.dot_general(wo_ref[...], sc.astype(jnp.bfloat16),
                                       (((1,), (1,)), ((), ())),
                                       preferred_element_type=jnp.float32)
            op_scr[pl.ds(b, 1), :, zsl, :] = part.astype(jnp.bfloat16)[None]
            ssum = ssum + jnp.sum(part, axis=(1, 2))
            ssq = ssq + jnp.sum(part * part, axis=(1, 2))
        st_scr[pl.ds(b, 1), 0, :] = ssum[None]
        st_scr[pl.ds(b, 1), 1, :] = ssq[None]

    @pl.when(s_id >= _CSTEPS)
    def _bn():
        n = float(_B * _S)
        ssum = st_scr[0, 0, :] + st_scr[1, 0, :]
        ssq = st_scr[0, 1, :] + st_scr[1, 1, :]
        mean = ssum / n
        var = ssq / n - mean * mean
        a = g_ref[0] * jax.lax.rsqrt(var + 1e-5)
        bb = b_ref[0] - mean * a
        op = op_scr[pl.ds(b, 1), :, pl.ds(d * _DC, _DC), :][0]
        y_ref[0] = (a[:, None, None] * op.astype(jnp.float32)
                    + bb[:, None, None] + f_ref[0])


def kernel(features, Wq, Wv, Wo, W_off, b_off, W_att, b_att, gamma, beta):
    f4 = features.reshape(_B, _C, _D, _HW)
    pe = jnp.asarray(_pe_dc())
    boff = b_off.reshape(1, _HP)
    batt = b_att.reshape(1, _HP)

    def f_map(s):
        return ((s // _ND) % _B, 0, s % _ND, 0)

    def y_map(s):
        c = s >= _CSTEPS
        return (jnp.where(c, (s // _ND) % _B, 0), 0, jnp.where(c, s % _ND, 0), 0)

    y = pl.pallas_call(
        _fused_kernel,
        grid=(_STEPS,),
        in_specs=[
            pl.BlockSpec((1, _C, _DC, _HW), f_map),
            pl.BlockSpec((_D, _C), lambda s: (0, 0)),
            pl.BlockSpec((_C, _C), lambda s: (0, 0)),
            pl.BlockSpec((_C, _C), lambda s: (0, 0)),
            pl.BlockSpec((_HP, _C), lambda s: (0, 0)),
            pl.BlockSpec((1, _HP), lambda s: (0, 0)),
            pl.BlockSpec((_HP, _C), lambda s: (0, 0)),
            pl.BlockSpec((1, _HP), lambda s: (0, 0)),
            pl.BlockSpec((_C, _C), lambda s: (0, 0)),
            pl.BlockSpec((1, _C), lambda s: (0, 0)),
            pl.BlockSpec((1, _C), lambda s: (0, 0)),
        ],
        out_specs=pl.BlockSpec((1, _C, _DC, _HW), y_map),
        out_shape=jax.ShapeDtypeStruct((_B, _C, _D, _HW), jnp.float32),
        scratch_shapes=[
            pltpu.VMEM((_C, _D, _HW), jnp.bfloat16),
            pltpu.VMEM((_D, _C), jnp.float32),
            pltpu.VMEM((_B, _C, _D, _HW), jnp.bfloat16),
            pltpu.VMEM((_B, 2, _C), jnp.float32),
        ],
    )(f4, pe, Wq.astype(jnp.bfloat16), Wv.astype(jnp.bfloat16),
      W_off, boff, W_att, batt, Wo.astype(jnp.bfloat16),
      gamma.reshape(1, _C), beta.reshape(1, _C))

    return y.reshape(_B, _C, _D, _H, _W)
